# Initial kernel scaffold; baseline (speedup 1.0000x reference)
import jax
import jax.numpy as jnp
from jax.experimental import pallas as pl


def kernel(node_emb, rel_emb, edge_mask, edge_index, edge_type, head_idx, tail_idx, rel_idx):
    raise NotImplementedError("write your pallas kernel here")


# speedup vs baseline: 20.3085x; 20.3085x over previous
import functools

import jax
import jax.numpy as jnp
from jax import lax
from jax.experimental import pallas as pl
from jax.experimental.pallas import tpu as pltpu
from jax.experimental.pallas import tpu_sc as plsc

N = 10000
E = 320000
D = 128
NPAD = 10240
NR = NPAD // 128
NC = 2
NS = 16
EA = E // NS
CA = 400
EW = E // (NC * NS)
CB = 80

_i32 = jnp.int32
_f32 = jnp.float32


def _rowcol(idx):
  return [lax.shift_right_logical(idx, 7), lax.bitwise_and(idx, 127)]


def _bfs_body(e0_hbm, e1_hbm, init_hbm, zero_hbm,
              all_out, loc_out,
              cur_v, all_v, new_v, tmp_v, e0c_v, e1c_v, init_v,
              stage_v, idvec_v, sh_hits):
  t = lax.axis_index("s")
  ones = jnp.ones((16,), _i32)
  lanes = jnp.arange(16, dtype=_i32)

  pltpu.sync_copy(zero_hbm, cur_v)
  pltpu.sync_copy(zero_hbm, all_v)
  pltpu.sync_copy(init_hbm, init_v)
  iv = init_v[...]
  plsc.store_scatter(cur_v, _rowcol(iv), ones)
  plsc.store_scatter(all_v, _rowcol(iv), ones)

  for g in range(NR // 16):
    idvec_v[pl.ds(16 * g, 16)] = lanes + 16 * g

  for _hop in range(2):
    pltpu.sync_copy(zero_hbm, new_v)

    @pl.when(t < 10)
    def _zero_hits():
      pltpu.sync_copy(zero_hbm.at[pl.ds(8 * t, 8)], sh_hits.at[pl.ds(8 * t, 8)])

    plsc.subcore_barrier()

    def chunk_body(ci, carry):
      base = t * EA + ci * CA
      pltpu.sync_copy(e0_hbm.at[pl.ds(base, CA)], e0c_v)
      pltpu.sync_copy(e1_hbm.at[pl.ds(base, CA)], e1c_v)
      for g in range(CA // 16):
        a = e0c_v[pl.ds(16 * g, 16)]
        b = e1c_v[pl.ds(16 * g, 16)]
        sm = plsc.load_gather(cur_v, _rowcol(a))
        dm = plsc.load_gather(cur_v, _rowcol(b))
        plsc.store_scatter(new_v, _rowcol(b), ones, mask=sm > 0)
        plsc.store_scatter(new_v, _rowcol(a), ones, mask=dm > 0)
      return carry

    lax.fori_loop(0, EA // CA, chunk_body, 0)
    pl.delay(200)
    pltpu.sync_copy(new_v, sh_hits.at[idvec_v], add=True)
    plsc.subcore_barrier()
    pltpu.sync_copy(sh_hits, tmp_v)

    def rb_body(i, carry):
      for j in range(8):
        h = tmp_v[i, pl.ds(16 * j, 16)]
        c = (h > 0).astype(_i32)
        cur_v[i, pl.ds(16 * j, 16)] = c
        all_v[i, pl.ds(16 * j, 16)] = lax.bitwise_or(all_v[i, pl.ds(16 * j, 16)], c)
      return carry

    lax.fori_loop(0, NR, rb_body, 0)
    plsc.subcore_barrier()

  def pre_body(i, acc):
    for j in range(8):
      acc = acc + all_v[i, pl.ds(16 * j, 16)]
    return acc

  acc0 = lax.fori_loop(0, 8 * t, pre_body, jnp.zeros((16,), _i32))
  run0 = jnp.full((16,), jnp.sum(acc0) - 1, _i32)

  def cs_body(i_loc, run):
    for j in range(8):
      x = all_v[8 * t + i_loc, pl.ds(16 * j, 16)]
      stage_v[i_loc, pl.ds(16 * j, 16)] = plsc.cumsum(x) + run
      run = run + jnp.full((16,), jnp.sum(x), _i32)
    return run

  nrows = jnp.where(t < 10, 8, 0)
  lax.fori_loop(0, nrows, cs_body, run0)
  pl.delay(200)

  @pl.when(t < 10)
  def _writeback():
    pltpu.sync_copy(stage_v, loc_out.at[pl.ds(8 * t, 8)])
    pltpu.sync_copy(all_v.at[pl.ds(8 * t, 8)], all_out.at[pl.ds(8 * t, 8)])


def _msg_body(node_hbm, e0_hbm, e1_hbm, mask_hbm, all_hbm, loc_hbm, zf_hbm,
              ss_out, sd_out, kp_out, part_out,
              an_v, loc_v, e0c_v, e1c_v, mc_v, rows_v,
              ssb_v, sdb_v, kpb_v, sh_agg, sem):
  cid = lax.axis_index("c")
  t = lax.axis_index("s")
  wid = cid * NS + t
  neg1 = jnp.full((16,), -1, _i32)

  pltpu.sync_copy(all_hbm, an_v)
  pltpu.sync_copy(loc_hbm, loc_v)
  pltpu.sync_copy(zf_hbm, rows_v)
  for k in range(8):
    pltpu.sync_copy(rows_v, sh_agg.at[pl.ds(t * 640 + k * CB, CB)])
  plsc.subcore_barrier()

  def chunk_body(ci, carry):
    base = wid * EW + ci * CB
    pltpu.sync_copy(e0_hbm.at[pl.ds(base, CB)], e0c_v)
    pltpu.sync_copy(e1_hbm.at[pl.ds(base, CB)], e1c_v)
    pltpu.sync_copy(mask_hbm.at[pl.ds(base, CB)], mc_v)
    pltpu.async_copy(node_hbm.at[e0c_v], rows_v, sem).wait()
    lanes = jnp.arange(16, dtype=_i32)
    zf = jnp.zeros((16,), _f32)
    for g in range(CB // 16):
      a = e0c_v[pl.ds(16 * g, 16)]
      b = e1c_v[pl.ds(16 * g, 16)]
      m = mc_v[pl.ds(16 * g, 16)]
      an0 = plsc.load_gather(an_v, _rowcol(a))
      an1 = plsc.load_gather(an_v, _rowcol(b))
      keep = lax.bitwise_and(an0, an1)
      kb = keep > 0
      l0 = plsc.load_gather(loc_v, _rowcol(a))
      l1 = plsc.load_gather(loc_v, _rowcol(b))
      ssb_v[pl.ds(16 * g, 16)] = jnp.where(kb, l0, neg1)
      sdb_v[pl.ds(16 * g, 16)] = jnp.where(kb, l1, neg1)
      kpb_v[pl.ds(16 * g, 16)] = keep
      w = keep.astype(_f32) / (1.0 + jnp.exp(-m))
      for r in range(16):
        ws = jnp.full((16,), jnp.sum(jnp.where(lanes == r, w, zf)), _f32)
        row = 16 * g + r
        for k in range(8):
          rows_v[row, pl.ds(16 * k, 16)] = rows_v[row, pl.ds(16 * k, 16)] * ws
    pl.delay(200)
    pltpu.sync_copy(rows_v, sh_agg.at[e1c_v], add=True)
    pltpu.sync_copy(ssb_v, ss_out.at[pl.ds(base, CB)])
    pltpu.sync_copy(sdb_v, sd_out.at[pl.ds(base, CB)])
    pltpu.sync_copy(kpb_v, kp_out.at[pl.ds(base, CB)])
    return carry

  lax.fori_loop(0, EW // CB, chunk_body, 0)
  plsc.subcore_barrier()

  @pl.when(t < 15)
  def _export_main():
    pltpu.sync_copy(sh_agg.at[pl.ds(t * 640, 640)],
                    part_out.at[pl.ds(cid * N + t * 640, 640)])

  @pl.when(t == 15)
  def _export_tail():
    pltpu.sync_copy(sh_agg.at[pl.ds(9600, 400)],
                    part_out.at[pl.ds(cid * N + 9600, 400)])


def _comb_body(p_ref, emb_ref, rel_ref, h_ref, t_ref, r_ref, agg_ref, sc_ref):
  agg_ref[...] = p_ref[0] + p_ref[1]
  h = h_ref[0]
  tt = t_ref[0]
  rr = r_ref[0]
  hrow = emb_ref[pl.ds(h, 1), :]
  trow = emb_ref[pl.ds(tt, 1), :]
  rrow = rel_ref[pl.ds(rr, 1), :]
  sc_ref[0, 0] = jnp.sum(hrow * rrow * trow)


_bfs_call = pl.kernel(
    _bfs_body,
    out_type=(
        jax.ShapeDtypeStruct((NR, 128), _i32),
        jax.ShapeDtypeStruct((NR, 128), _i32),
    ),
    mesh=plsc.VectorSubcoreMesh(core_axis_name="c", subcore_axis_name="s"),
    compiler_params=pltpu.CompilerParams(needs_layout_passes=False),
    scratch_types=(
        pltpu.VMEM((NR, 128), _i32),
        pltpu.VMEM((NR, 128), _i32),
        pltpu.VMEM((NR, 128), _i32),
        pltpu.VMEM((NR, 128), _i32),
        pltpu.VMEM((CA,), _i32),
        pltpu.VMEM((CA,), _i32),
        pltpu.VMEM((16,), _i32),
        pltpu.VMEM((8, 128), _i32),
        pltpu.VMEM((NR,), _i32),
        pltpu.VMEM_SHARED((NR, 128), _i32),
    ),
)

_msg_call = pl.kernel(
    _msg_body,
    out_type=(
        jax.ShapeDtypeStruct((E,), _i32),
        jax.ShapeDtypeStruct((E,), _i32),
        jax.ShapeDtypeStruct((E,), _i32),
        jax.ShapeDtypeStruct((NC * N, D), _f32),
    ),
    mesh=plsc.VectorSubcoreMesh(core_axis_name="c", subcore_axis_name="s"),
    compiler_params=pltpu.CompilerParams(needs_layout_passes=False),
    scratch_types=(
        pltpu.VMEM((NR, 128), _i32),
        pltpu.VMEM((NR, 128), _i32),
        pltpu.VMEM((CB,), _i32),
        pltpu.VMEM((CB,), _i32),
        pltpu.VMEM((CB,), _f32),
        pltpu.VMEM((CB, D), _f32),
        pltpu.VMEM((CB,), _i32),
        pltpu.VMEM((CB,), _i32),
        pltpu.VMEM((CB,), _i32),
        pltpu.VMEM_SHARED((NPAD, D), _f32),
        pltpu.SemaphoreType.DMA,
    ),
)

_comb_call = pl.pallas_call(
    _comb_body,
    out_shape=(
        jax.ShapeDtypeStruct((N, D), _f32),
        jax.ShapeDtypeStruct((1, 1), _f32),
    ),
    in_specs=[
        pl.BlockSpec(memory_space=pltpu.VMEM),
        pl.BlockSpec(memory_space=pltpu.VMEM),
        pl.BlockSpec(memory_space=pltpu.VMEM),
        pl.BlockSpec(memory_space=pltpu.SMEM),
        pl.BlockSpec(memory_space=pltpu.SMEM),
        pl.BlockSpec(memory_space=pltpu.SMEM),
    ],
    out_specs=(
        pl.BlockSpec(memory_space=pltpu.VMEM),
        pl.BlockSpec(memory_space=pltpu.SMEM),
    ),
)


def kernel(node_emb, rel_emb, edge_mask, edge_index, edge_type,
           head_idx, tail_idx, rel_idx):
  del edge_type
  e0 = edge_index[0]
  e1 = edge_index[1]
  h = jnp.asarray(head_idx, _i32)
  t = jnp.asarray(tail_idx, _i32)
  r = jnp.asarray(rel_idx, _i32)
  init_idx = jnp.concatenate(
      [h[None], t[None], jnp.broadcast_to(h[None], (14,))])
  zero_i = jnp.zeros((NR, 128), _i32)
  zero_f = jnp.zeros((CB, D), _f32)

  all2d, loc2d = _bfs_call(e0, e1, init_idx, zero_i)
  sub_src, sub_dst, keep_i, parts = _msg_call(
      node_emb, e0, e1, edge_mask, all2d, loc2d, zero_f)
  p = parts.reshape(NC, N, D)
  agg, score = _comb_call(p, node_emb, rel_emb, h[None], t[None], r[None])
  return (agg, score.reshape(()), sub_src, sub_dst, keep_i.astype(bool))


# speedup vs baseline: 35.8179x; 1.7637x over previous
import functools

import jax
import jax.numpy as jnp
from jax import lax
from jax.experimental import pallas as pl
from jax.experimental.pallas import tpu as pltpu
from jax.experimental.pallas import tpu_sc as plsc

N = 10000
E = 320000
D = 128
NPAD = 10240
NR = NPAD // 128
NC = 2
NS = 16
EA = E // NS
CA = 2000
EW = E // (NC * NS)
CB = 80
NCH = EW // CB
BCH = 25
NB = NCH // BCH

_i32 = jnp.int32
_f32 = jnp.float32


def _rowcol(idx):
  return [lax.shift_right_logical(idx, 7), lax.bitwise_and(idx, 127)]


def _bfs_body(e0_hbm, e1_hbm, init_hbm, zero_hbm,
              enc_out,
              cur_v, all_v, new_v, tmp_v, e0c_v, e1c_v, init_v,
              stage_v, idvec_v, sh_hits):
  t = lax.axis_index("s")
  ones = jnp.ones((16,), _i32)
  lanes = jnp.arange(16, dtype=_i32)

  pltpu.sync_copy(zero_hbm, cur_v)
  pltpu.sync_copy(zero_hbm, all_v)
  pltpu.sync_copy(init_hbm, init_v)
  iv = init_v[...]
  plsc.store_scatter(cur_v, _rowcol(iv), ones)
  plsc.store_scatter(all_v, _rowcol(iv), ones)

  for g in range(NR // 16):
    idvec_v[pl.ds(16 * g, 16)] = lanes + 16 * g

  for _hop in range(2):
    pltpu.sync_copy(zero_hbm, new_v)

    @pl.when(t < 10)
    def _zero_hits():
      pltpu.sync_copy(zero_hbm.at[pl.ds(8 * t, 8)], sh_hits.at[pl.ds(8 * t, 8)])

    plsc.subcore_barrier()

    def chunk_body(ci, carry):
      base = t * EA + ci * CA
      pltpu.sync_copy(e0_hbm.at[pl.ds(base, CA)], e0c_v)
      pltpu.sync_copy(e1_hbm.at[pl.ds(base, CA)], e1c_v)
      for g in range(CA // 16):
        a = e0c_v[pl.ds(16 * g, 16)]
        b = e1c_v[pl.ds(16 * g, 16)]
        sm = plsc.load_gather(cur_v, _rowcol(a))
        dm = plsc.load_gather(cur_v, _rowcol(b))
        plsc.store_scatter(new_v, _rowcol(b), ones, mask=sm > 0)
        plsc.store_scatter(new_v, _rowcol(a), ones, mask=dm > 0)
      return carry

    lax.fori_loop(0, EA // CA, chunk_body, 0)
    pl.delay(200)
    pltpu.sync_copy(new_v, sh_hits.at[idvec_v], add=True)
    plsc.subcore_barrier()
    pltpu.sync_copy(sh_hits, tmp_v)

    def rb_body(i, carry):
      for j in range(8):
        h = tmp_v[i, pl.ds(16 * j, 16)]
        c = (h > 0).astype(_i32)
        cur_v[i, pl.ds(16 * j, 16)] = c
        all_v[i, pl.ds(16 * j, 16)] = lax.bitwise_or(all_v[i, pl.ds(16 * j, 16)], c)
      return carry

    lax.fori_loop(0, NR, rb_body, 0)
    plsc.subcore_barrier()

  def pre_body(i, acc):
    for j in range(8):
      acc = acc + all_v[i, pl.ds(16 * j, 16)]
    return acc

  acc0 = lax.fori_loop(0, 8 * t, pre_body, jnp.zeros((16,), _i32))
  run0 = jnp.full((16,), jnp.sum(acc0), _i32)

  def cs_body(i_loc, run):
    for j in range(8):
      x = all_v[8 * t + i_loc, pl.ds(16 * j, 16)]
      pre = plsc.cumsum(x) + run
      stage_v[i_loc, pl.ds(16 * j, 16)] = lax.bitwise_or(
          lax.shift_left(pre, 1), x)
      run = run + jnp.full((16,), jnp.sum(x), _i32)
    return run

  nrows = jnp.where(t < 10, 8, 0)
  lax.fori_loop(0, nrows, cs_body, run0)
  pl.delay(200)

  @pl.when(t < 10)
  def _writeback():
    pltpu.sync_copy(stage_v, enc_out.at[pl.ds(8 * t, 8)])


def _msg_body(node_hbm, e0r_hbm, e1r_hbm, mr_hbm, enc_hbm, zf_hbm,
              pk_out, part_out,
              enc_v, e0t_v, e1t_v, mt_v,
              rows_a, rows_b, sh_agg, semg_a, semg_b):
  cid = lax.axis_index("c")
  t = lax.axis_index("s")
  wid = cid * NS + t
  neg1 = jnp.full((16,), -1, _i32)
  ones = jnp.ones((16,), _i32)
  lanes = jnp.arange(16, dtype=_i32)
  zf = jnp.zeros((16,), _f32)

  pltpu.sync_copy(enc_hbm, enc_v)

  @pl.when(t < 10)
  def _zero_agg():
    pltpu.sync_copy(zf_hbm, sh_agg.at[pl.ds(1000 * t, 1000)])

  plsc.subcore_barrier()

  def issue_gather(ci, rows_v, sem):
    return pltpu.async_copy(node_hbm.at[e0t_v.at[ci]], rows_v, sem)

  def wait_gather(ci, rows_v, sem):
    pltpu.make_async_copy(node_hbm.at[e0t_v.at[ci]], rows_v, sem).wait()

  def compute_scale(ci, rows_v):
    keeps = []
    for g in range(CB // 16):
      a = e0t_v[ci, pl.ds(16 * g, 16)]
      b = e1t_v[ci, pl.ds(16 * g, 16)]
      m = plsc.bitcast(mt_v[ci, pl.ds(16 * g, 16)], _f32)
      enc0 = plsc.load_gather(enc_v, _rowcol(a))
      enc1 = plsc.load_gather(enc_v, _rowcol(b))
      keep = lax.bitwise_and(lax.bitwise_and(enc0, enc1), ones)
      keeps.append((enc0, enc1, keep))
      w = keep.astype(_f32) / (1.0 + jnp.exp(-m))

      @plsc.parallel_loop(0, 16, 1, unroll=2)
      def _scale(r):
        ws = jnp.full((16,), jnp.sum(jnp.where(lanes == r, w, zf)), _f32)
        for k in range(8):
          rows_v[16 * g + r, pl.ds(16 * k, 16)] = (
              rows_v[16 * g + r, pl.ds(16 * k, 16)] * ws)

    return keeps

  def emit_outputs(ci, keeps):
    kbit = jnp.full((16,), 1 << 28, _i32)
    zi = jnp.zeros((16,), _i32)
    for g in range(CB // 16):
      enc0, enc1, keep = keeps[g]
      kb = keep > 0
      hi = lax.bitwise_or(
          lax.bitwise_or(lax.shift_right_logical(enc0, 1),
                         lax.shift_left(lax.shift_right_logical(enc1, 1), 14)),
          kbit)
      mt_v[ci, pl.ds(16 * g, 16)] = jnp.where(kb, hi, zi)

  def scatter_add(ci, rows_v):
    pl.delay(100)
    pltpu.sync_copy(rows_v, sh_agg.at[e1t_v.at[ci]], add=True)

  def batch_body(b, carry):
    pltpu.sync_copy(e0r_hbm.at[wid, b], e0t_v)
    pltpu.sync_copy(e1r_hbm.at[wid, b], e1t_v)
    pltpu.sync_copy(mr_hbm.at[wid, b], mt_v)
    issue_gather(0, rows_a, semg_a)

    def pair_body(i, carry2):
      c0 = 2 * i
      c1 = 2 * i + 1
      wait_gather(c0, rows_a, semg_a)
      keeps = compute_scale(c0, rows_a)
      issue_gather(c1, rows_b, semg_b)
      emit_outputs(c0, keeps)
      scatter_add(c0, rows_a)
      wait_gather(c1, rows_b, semg_b)
      keeps = compute_scale(c1, rows_b)
      issue_gather(c1 + 1, rows_a, semg_a)
      emit_outputs(c1, keeps)
      scatter_add(c1, rows_b)
      return carry2

    lax.fori_loop(0, BCH // 2, pair_body, 0)
    ct = BCH - 1
    wait_gather(ct, rows_a, semg_a)
    keeps = compute_scale(ct, rows_a)
    emit_outputs(ct, keeps)
    pl.delay(200)
    scatter_add(ct, rows_a)
    pltpu.sync_copy(mt_v, pk_out.at[wid, b])
    return carry

  lax.fori_loop(0, NB, batch_body, 0)
  plsc.subcore_barrier()

  @pl.when(t < 10)
  def _export():
    pltpu.sync_copy(sh_agg.at[pl.ds(1000 * t, 1000)],
                    part_out.at[pl.ds(cid * N + 1000 * t, 1000)])


def _comb_body(p_ref, emb_ref, rel_ref, h_ref, t_ref, r_ref, agg_ref, sc_ref):
  agg_ref[...] = p_ref[0] + p_ref[1]
  h = h_ref[0]
  tt = t_ref[0]
  rr = r_ref[0]
  hrow = emb_ref[pl.ds(h, 1), :]
  trow = emb_ref[pl.ds(tt, 1), :]
  rrow = rel_ref[pl.ds(rr, 1), :]
  sc_ref[0, 0] = jnp.sum(hrow * rrow * trow)


_bfs_call = pl.kernel(
    _bfs_body,
    out_type=(
        jax.ShapeDtypeStruct((NR, 128), _i32),
    ),
    mesh=plsc.VectorSubcoreMesh(core_axis_name="c", subcore_axis_name="s"),
    compiler_params=pltpu.CompilerParams(needs_layout_passes=False),
    scratch_types=(
        pltpu.VMEM((NR, 128), _i32),
        pltpu.VMEM((NR, 128), _i32),
        pltpu.VMEM((NR, 128), _i32),
        pltpu.VMEM((NR, 128), _i32),
        pltpu.VMEM((CA,), _i32),
        pltpu.VMEM((CA,), _i32),
        pltpu.VMEM((16,), _i32),
        pltpu.VMEM((8, 128), _i32),
        pltpu.VMEM((NR,), _i32),
        pltpu.VMEM_SHARED((NR, 128), _i32),
    ),
)

_msg_call = pl.kernel(
    _msg_body,
    out_type=(
        jax.ShapeDtypeStruct((NC * NS, NB, BCH, CB), _i32),
        jax.ShapeDtypeStruct((NC * N, D), _f32),
    ),
    mesh=plsc.VectorSubcoreMesh(core_axis_name="c", subcore_axis_name="s"),
    compiler_params=pltpu.CompilerParams(needs_layout_passes=False),
    scratch_types=(
        pltpu.VMEM((NR, 128), _i32),
        pltpu.VMEM((BCH, CB), _i32),
        pltpu.VMEM((BCH, CB), _i32),
        pltpu.VMEM((BCH, CB), _i32),
        pltpu.VMEM((CB, D), _f32),
        pltpu.VMEM((CB, D), _f32),
        pltpu.VMEM_SHARED((N, D), _f32),
        pltpu.SemaphoreType.DMA,
        pltpu.SemaphoreType.DMA,
    ),
)

_comb_call = pl.pallas_call(
    _comb_body,
    out_shape=(
        jax.ShapeDtypeStruct((N, D), _f32),
        jax.ShapeDtypeStruct((1, 1), _f32),
    ),
    in_specs=[
        pl.BlockSpec(memory_space=pltpu.VMEM),
        pl.BlockSpec(memory_space=pltpu.VMEM),
        pl.BlockSpec(memory_space=pltpu.VMEM),
        pl.BlockSpec(memory_space=pltpu.SMEM),
        pl.BlockSpec(memory_space=pltpu.SMEM),
        pl.BlockSpec(memory_space=pltpu.SMEM),
    ],
    out_specs=(
        pl.BlockSpec(memory_space=pltpu.VMEM),
        pl.BlockSpec(memory_space=pltpu.SMEM),
    ),
)


def kernel(node_emb, rel_emb, edge_mask, edge_index, edge_type,
           head_idx, tail_idx, rel_idx):
  del edge_type
  e0 = edge_index[0]
  e1 = edge_index[1]
  h = jnp.asarray(head_idx, _i32)
  t = jnp.asarray(tail_idx, _i32)
  r = jnp.asarray(rel_idx, _i32)
  init_idx = jnp.concatenate(
      [h[None], t[None], jnp.broadcast_to(h[None], (14,))])
  zero_i = jnp.zeros((NR, 128), _i32)
  zero_f = jnp.zeros((1000, D), _f32)

  (enc2d,) = _bfs_call(e0, e1, init_idx, zero_i)
  e0r = e0.reshape(NC * NS, NB, BCH, CB)
  e1r = e1.reshape(NC * NS, NB, BCH, CB)
  mr = lax.bitcast_convert_type(edge_mask, _i32).reshape(NC * NS, NB, BCH, CB)
  pk, parts = _msg_call(node_emb, e0r, e1r, mr, enc2d, zero_f)
  p = parts.reshape(NC, N, D)
  agg, score = _comb_call(p, node_emb, rel_emb, h[None], t[None], r[None])
  pk = pk.reshape(E)
  sub_src = (pk & 0x3FFF) - 1
  sub_dst = ((pk >> 14) & 0x3FFF) - 1
  keep = (pk >> 28).astype(bool)
  return (agg, score.reshape(()), sub_src, sub_dst, keep)


# speedup vs baseline: 38.4727x; 1.0741x over previous
import functools

import jax
import jax.numpy as jnp
from jax import lax
from jax.experimental import pallas as pl
from jax.experimental.pallas import tpu as pltpu
from jax.experimental.pallas import tpu_sc as plsc

N = 10000
E = 320000
D = 128
NPAD = 10240
NR = NPAD // 128
NC = 2
NS = 16
EA = E // NS
CA = 2000
EW = E // (NC * NS)
CB = 80
NCH = EW // CB
BCH = 25
NB = NCH // BCH

_i32 = jnp.int32
_f32 = jnp.float32


def _rowcol(idx):
  return [lax.shift_right_logical(idx, 7), lax.bitwise_and(idx, 127)]


def _bfs_body(e0_hbm, e1_hbm, init_hbm, zero_hbm,
              enc_out,
              cur_v, all_v, new_v, tmp_v, e0c_v, e1c_v, init_v,
              stage_v, idvec_v, sh_hits):
  t = lax.axis_index("s")
  ones = jnp.ones((16,), _i32)
  lanes = jnp.arange(16, dtype=_i32)

  pltpu.sync_copy(zero_hbm, cur_v)
  pltpu.sync_copy(zero_hbm, all_v)
  pltpu.sync_copy(init_hbm, init_v)
  iv = init_v[...]
  plsc.store_scatter(cur_v, _rowcol(iv), ones)
  plsc.store_scatter(all_v, _rowcol(iv), ones)

  for g in range(NR // 16):
    idvec_v[pl.ds(16 * g, 16)] = lanes + 16 * g

  pltpu.sync_copy(e0_hbm.at[pl.ds(t * EA, EA)], e0c_v)
  pltpu.sync_copy(e1_hbm.at[pl.ds(t * EA, EA)], e1c_v)

  for _hop in range(2):
    pltpu.sync_copy(zero_hbm, new_v)

    @pl.when(t < 10)
    def _zero_hits():
      pltpu.sync_copy(zero_hbm.at[pl.ds(8 * t, 8)], sh_hits.at[pl.ds(8 * t, 8)])

    plsc.subcore_barrier()

    def chunk_body(ci, carry):
      for g in range(5):
        gi = 80 * ci + 16 * g
        a = e0c_v[pl.ds(gi, 16)]
        b = e1c_v[pl.ds(gi, 16)]
        sm = plsc.load_gather(cur_v, _rowcol(a))
        dm = plsc.load_gather(cur_v, _rowcol(b))
        plsc.store_scatter(new_v, _rowcol(b), ones, mask=sm > 0)
        plsc.store_scatter(new_v, _rowcol(a), ones, mask=dm > 0)
      return carry

    lax.fori_loop(0, EA // 80, chunk_body, 0)
    pl.delay(200)
    pltpu.sync_copy(new_v, sh_hits.at[idvec_v], add=True)
    plsc.subcore_barrier()
    pltpu.sync_copy(sh_hits, tmp_v)

    def rb_body(i, carry):
      for j in range(8):
        h = tmp_v[i, pl.ds(16 * j, 16)]
        c = (h > 0).astype(_i32)
        cur_v[i, pl.ds(16 * j, 16)] = c
        all_v[i, pl.ds(16 * j, 16)] = lax.bitwise_or(all_v[i, pl.ds(16 * j, 16)], c)
      return carry

    lax.fori_loop(0, NR, rb_body, 0)
    plsc.subcore_barrier()

  def pre_body(i, acc):
    for j in range(8):
      acc = acc + all_v[i, pl.ds(16 * j, 16)]
    return acc

  acc0 = lax.fori_loop(0, 8 * t, pre_body, jnp.zeros((16,), _i32))
  run0 = jnp.full((16,), jnp.sum(acc0), _i32)

  def cs_body(i_loc, run):
    for j in range(8):
      x = all_v[8 * t + i_loc, pl.ds(16 * j, 16)]
      pre = plsc.cumsum(x) + run
      stage_v[i_loc, pl.ds(16 * j, 16)] = lax.bitwise_or(
          lax.shift_left(pre, 1), x)
      run = run + jnp.full((16,), jnp.sum(x), _i32)
    return run

  nrows = jnp.where(t < 10, 8, 0)
  lax.fori_loop(0, nrows, cs_body, run0)
  pl.delay(200)

  @pl.when(t < 10)
  def _writeback():
    pltpu.sync_copy(stage_v, enc_out.at[pl.ds(8 * t, 8)])


def _msg_body(node_hbm, e0r_hbm, e1r_hbm, mr_hbm, enc_hbm, zf_hbm,
              pk_out, part_out,
              enc_v, e0t_v, e1t_v, mt_v,
              rows_a, rows_b, sh_agg, semg_a, semg_b, sems_a, sems_b):
  cid = lax.axis_index("c")
  t = lax.axis_index("s")
  wid = cid * NS + t
  neg1 = jnp.full((16,), -1, _i32)
  ones = jnp.ones((16,), _i32)
  lanes = jnp.arange(16, dtype=_i32)
  zf = jnp.zeros((16,), _f32)

  pltpu.sync_copy(enc_hbm, enc_v)

  @pl.when(t < 10)
  def _zero_agg():
    pltpu.sync_copy(zf_hbm, sh_agg.at[pl.ds(1000 * t, 1000)])

  plsc.subcore_barrier()

  def issue_gather(ci, rows_v, sem):
    return pltpu.async_copy(node_hbm.at[e0t_v.at[ci]], rows_v, sem)

  def wait_gather(ci, rows_v, sem):
    pltpu.make_async_copy(node_hbm.at[e0t_v.at[ci]], rows_v, sem).wait()

  def compute_scale(ci, rows_v):
    keeps = []
    for g in range(CB // 16):
      a = e0t_v[ci, pl.ds(16 * g, 16)]
      b = e1t_v[ci, pl.ds(16 * g, 16)]
      m = plsc.bitcast(mt_v[ci, pl.ds(16 * g, 16)], _f32)
      enc0 = plsc.load_gather(enc_v, _rowcol(a))
      enc1 = plsc.load_gather(enc_v, _rowcol(b))
      keep = lax.bitwise_and(lax.bitwise_and(enc0, enc1), ones)
      keeps.append((enc0, enc1, keep))
      w = keep.astype(_f32) / (1.0 + jnp.exp(-m))

      @plsc.parallel_loop(0, 16, 1, unroll=2)
      def _scale(r):
        ws = jnp.full((16,), jnp.sum(jnp.where(lanes == r, w, zf)), _f32)
        for k in range(8):
          rows_v[16 * g + r, pl.ds(16 * k, 16)] = (
              rows_v[16 * g + r, pl.ds(16 * k, 16)] * ws)

    return keeps

  def emit_outputs(ci, keeps):
    kbit = jnp.full((16,), 1 << 28, _i32)
    zi = jnp.zeros((16,), _i32)
    for g in range(CB // 16):
      enc0, enc1, keep = keeps[g]
      kb = keep > 0
      hi = lax.bitwise_or(
          lax.bitwise_or(lax.shift_right_logical(enc0, 1),
                         lax.shift_left(lax.shift_right_logical(enc1, 1), 14)),
          kbit)
      mt_v[ci, pl.ds(16 * g, 16)] = jnp.where(kb, hi, zi)

  def issue_scatter(ci, rows_v, sem):
    pl.delay(100)
    return pltpu.async_copy(rows_v, sh_agg.at[e1t_v.at[ci]], sem, add=True)

  def wait_scatter(ci, rows_v, sem):
    pltpu.make_async_copy(rows_v, sh_agg.at[e1t_v.at[ci]], sem).wait()

  def batch_body(b, carry):
    pltpu.sync_copy(e0r_hbm.at[wid, b], e0t_v)
    pltpu.sync_copy(e1r_hbm.at[wid, b], e1t_v)
    pltpu.sync_copy(mr_hbm.at[wid, b], mt_v)
    issue_gather(0, rows_a, semg_a)

    def pair_body(i, carry2):
      c0 = 2 * i
      c1 = 2 * i + 1
      wait_gather(c0, rows_a, semg_a)

      @pl.when(i > 0)
      def _drain_b():
        wait_scatter(c0 - 1, rows_b, sems_b)

      issue_gather(c1, rows_b, semg_b)
      keeps = compute_scale(c0, rows_a)
      emit_outputs(c0, keeps)
      issue_scatter(c0, rows_a, sems_a)
      wait_gather(c1, rows_b, semg_b)
      wait_scatter(c0, rows_a, sems_a)
      issue_gather(c0 + 2, rows_a, semg_a)
      keeps = compute_scale(c1, rows_b)
      emit_outputs(c1, keeps)
      issue_scatter(c1, rows_b, sems_b)
      return carry2

    lax.fori_loop(0, BCH // 2, pair_body, 0)
    ct = BCH - 1
    wait_gather(ct, rows_a, semg_a)
    wait_scatter(ct - 1, rows_b, sems_b)
    keeps = compute_scale(ct, rows_a)
    emit_outputs(ct, keeps)
    pl.delay(200)
    pltpu.sync_copy(rows_a, sh_agg.at[e1t_v.at[ct]], add=True)
    pltpu.sync_copy(mt_v, pk_out.at[wid, b])
    return carry

  lax.fori_loop(0, NB, batch_body, 0)
  plsc.subcore_barrier()

  @pl.when(t < 10)
  def _export():
    pltpu.sync_copy(sh_agg.at[pl.ds(1000 * t, 1000)],
                    part_out.at[pl.ds(cid * N + 1000 * t, 1000)])


def _comb_body(p_ref, emb_ref, rel_ref, h_ref, t_ref, r_ref, agg_ref, sc_ref):
  agg_ref[...] = p_ref[0] + p_ref[1]
  h = h_ref[0]
  tt = t_ref[0]
  rr = r_ref[0]
  hrow = emb_ref[pl.ds(h, 1), :]
  trow = emb_ref[pl.ds(tt, 1), :]
  rrow = rel_ref[pl.ds(rr, 1), :]
  sc_ref[0, 0] = jnp.sum(hrow * rrow * trow)


_bfs_call = pl.kernel(
    _bfs_body,
    out_type=(
        jax.ShapeDtypeStruct((NR, 128), _i32),
    ),
    mesh=plsc.VectorSubcoreMesh(core_axis_name="c", subcore_axis_name="s"),
    compiler_params=pltpu.CompilerParams(needs_layout_passes=False),
    scratch_types=(
        pltpu.VMEM((NR, 128), _i32),
        pltpu.VMEM((NR, 128), _i32),
        pltpu.VMEM((NR, 128), _i32),
        pltpu.VMEM((NR, 128), _i32),
        pltpu.VMEM((EA,), _i32),
        pltpu.VMEM((EA,), _i32),
        pltpu.VMEM((16,), _i32),
        pltpu.VMEM((8, 128), _i32),
        pltpu.VMEM((NR,), _i32),
        pltpu.VMEM_SHARED((NR, 128), _i32),
    ),
)

_msg_call = pl.kernel(
    _msg_body,
    out_type=(
        jax.ShapeDtypeStruct((NC * NS, NB, BCH, CB), _i32),
        jax.ShapeDtypeStruct((NC * N, D), _f32),
    ),
    mesh=plsc.VectorSubcoreMesh(core_axis_name="c", subcore_axis_name="s"),
    compiler_params=pltpu.CompilerParams(needs_layout_passes=False),
    scratch_types=(
        pltpu.VMEM((NR, 128), _i32),
        pltpu.VMEM((BCH, CB), _i32),
        pltpu.VMEM((BCH, CB), _i32),
        pltpu.VMEM((BCH, CB), _i32),
        pltpu.VMEM((CB, D), _f32),
        pltpu.VMEM((CB, D), _f32),
        pltpu.VMEM_SHARED((N, D), _f32),
        pltpu.SemaphoreType.DMA,
        pltpu.SemaphoreType.DMA,
        pltpu.SemaphoreType.DMA,
        pltpu.SemaphoreType.DMA,
    ),
)

_comb_call = pl.pallas_call(
    _comb_body,
    out_shape=(
        jax.ShapeDtypeStruct((N, D), _f32),
        jax.ShapeDtypeStruct((1, 1), _f32),
    ),
    in_specs=[
        pl.BlockSpec(memory_space=pltpu.VMEM),
        pl.BlockSpec(memory_space=pltpu.VMEM),
        pl.BlockSpec(memory_space=pltpu.VMEM),
        pl.BlockSpec(memory_space=pltpu.SMEM),
        pl.BlockSpec(memory_space=pltpu.SMEM),
        pl.BlockSpec(memory_space=pltpu.SMEM),
    ],
    out_specs=(
        pl.BlockSpec(memory_space=pltpu.VMEM),
        pl.BlockSpec(memory_space=pltpu.SMEM),
    ),
)


def kernel(node_emb, rel_emb, edge_mask, edge_index, edge_type,
           head_idx, tail_idx, rel_idx):
  del edge_type
  e0 = edge_index[0]
  e1 = edge_index[1]
  h = jnp.asarray(head_idx, _i32)
  t = jnp.asarray(tail_idx, _i32)
  r = jnp.asarray(rel_idx, _i32)
  init_idx = jnp.concatenate(
      [h[None], t[None], jnp.broadcast_to(h[None], (14,))])
  zero_i = jnp.zeros((NR, 128), _i32)
  zero_f = jnp.zeros((1000, D), _f32)

  (enc2d,) = _bfs_call(e0, e1, init_idx, zero_i)
  e0r = e0.reshape(NC * NS, NB, BCH, CB)
  e1r = e1.reshape(NC * NS, NB, BCH, CB)
  mr = lax.bitcast_convert_type(edge_mask, _i32).reshape(NC * NS, NB, BCH, CB)
  pk, parts = _msg_call(node_emb, e0r, e1r, mr, enc2d, zero_f)
  p = parts.reshape(NC, N, D)
  agg, score = _comb_call(p, node_emb, rel_emb, h[None], t[None], r[None])
  pk = pk.reshape(E)
  sub_src = (pk & 0x3FFF) - 1
  sub_dst = ((pk >> 14) & 0x3FFF) - 1
  keep = (pk >> 28).astype(bool)
  return (agg, score.reshape(()), sub_src, sub_dst, keep)


# speedup vs baseline: 38.9191x; 1.0116x over previous
import functools

import jax
import jax.numpy as jnp
from jax import lax
from jax.experimental import pallas as pl
from jax.experimental.pallas import tpu as pltpu
from jax.experimental.pallas import tpu_sc as plsc

N = 10000
E = 320000
D = 128
NPAD = 10240
NR = NPAD // 128
NC = 2
NS = 16
EA = E // NS
CA = 2000
EW = E // (NC * NS)
CB = 80
NCH = EW // CB
BCH = 25
NB = NCH // BCH

_i32 = jnp.int32
_f32 = jnp.float32


def _rowcol(idx):
  return [lax.shift_right_logical(idx, 7), lax.bitwise_and(idx, 127)]


def _bfs_body(e0_hbm, e1_hbm, init_hbm, zero_hbm,
              enc_out,
              cur_v, all_v, new_v, tmp_v, e0c_v, e1c_v, init_v,
              stage_v, idvec_v, sh_hits):
  t = lax.axis_index("s")
  ones = jnp.ones((16,), _i32)
  lanes = jnp.arange(16, dtype=_i32)

  pltpu.sync_copy(zero_hbm, cur_v)
  pltpu.sync_copy(zero_hbm, all_v)
  pltpu.sync_copy(init_hbm, init_v)
  iv = init_v[...]
  plsc.store_scatter(cur_v, _rowcol(iv), ones)
  plsc.store_scatter(all_v, _rowcol(iv), ones)

  for g in range(NR // 16):
    idvec_v[pl.ds(16 * g, 16)] = lanes + 16 * g

  pltpu.sync_copy(e0_hbm.at[pl.ds(t * EA, EA)], e0c_v)
  pltpu.sync_copy(e1_hbm.at[pl.ds(t * EA, EA)], e1c_v)

  for _hop in range(2):
    pltpu.sync_copy(zero_hbm, new_v)

    @pl.when(t < 10)
    def _zero_hits():
      pltpu.sync_copy(zero_hbm.at[pl.ds(8 * t, 8)], sh_hits.at[pl.ds(8 * t, 8)])

    plsc.subcore_barrier()

    def chunk_body(ci, carry):
      for g in range(5):
        gi = 80 * ci + 16 * g
        a = e0c_v[pl.ds(gi, 16)]
        b = e1c_v[pl.ds(gi, 16)]
        sm = plsc.load_gather(cur_v, _rowcol(a))
        dm = plsc.load_gather(cur_v, _rowcol(b))
        plsc.store_scatter(new_v, _rowcol(b), ones, mask=sm > 0)
        plsc.store_scatter(new_v, _rowcol(a), ones, mask=dm > 0)
      return carry

    lax.fori_loop(0, EA // 80, chunk_body, 0)
    pl.delay(200)
    pltpu.sync_copy(new_v, sh_hits.at[idvec_v], add=True)
    plsc.subcore_barrier()
    pltpu.sync_copy(sh_hits, tmp_v)

    def rb_body(i, carry):
      for j in range(8):
        h = tmp_v[i, pl.ds(16 * j, 16)]
        c = (h > 0).astype(_i32)
        cur_v[i, pl.ds(16 * j, 16)] = c
        all_v[i, pl.ds(16 * j, 16)] = lax.bitwise_or(all_v[i, pl.ds(16 * j, 16)], c)
      return carry

    lax.fori_loop(0, NR, rb_body, 0)
    plsc.subcore_barrier()

  def pre_body(i, acc):
    for j in range(8):
      acc = acc + all_v[i, pl.ds(16 * j, 16)]
    return acc

  acc0 = lax.fori_loop(0, 8 * t, pre_body, jnp.zeros((16,), _i32))
  run0 = jnp.full((16,), jnp.sum(acc0), _i32)

  def cs_body(i_loc, run):
    for j in range(8):
      x = all_v[8 * t + i_loc, pl.ds(16 * j, 16)]
      pre = plsc.cumsum(x) + run
      stage_v[i_loc, pl.ds(16 * j, 16)] = lax.bitwise_or(
          lax.shift_left(pre, 1), x)
      run = run + jnp.full((16,), jnp.sum(x), _i32)
    return run

  nrows = jnp.where(t < 10, 8, 0)
  lax.fori_loop(0, nrows, cs_body, run0)
  pl.delay(200)

  @pl.when(t < 10)
  def _writeback():
    pltpu.sync_copy(stage_v, enc_out.at[pl.ds(8 * t, 8)])


def _msg_body(node_hbm, e0r_hbm, e1r_hbm, mr_hbm, enc_hbm, zf_hbm,
              pk_out, part_out,
              enc_v, e0t_v, e1t_v, mt_v,
              rows_a, rows_b, sh_agg, semg_a, semg_b, sems_a, sems_b):
  cid = lax.axis_index("c")
  t = lax.axis_index("s")
  wid = cid * NS + t
  neg1 = jnp.full((16,), -1, _i32)
  ones = jnp.ones((16,), _i32)
  lanes = jnp.arange(16, dtype=_i32)
  zf = jnp.zeros((16,), _f32)

  pltpu.sync_copy(enc_hbm, enc_v)

  @pl.when(t < 10)
  def _zero_agg():
    pltpu.sync_copy(zf_hbm, sh_agg.at[pl.ds(1000 * t, 1000)])

  plsc.subcore_barrier()

  def issue_gather(ci, rows_v, sem):
    return pltpu.async_copy(node_hbm.at[e0t_v.at[ci]], rows_v, sem)

  def wait_gather(ci, rows_v, sem):
    pltpu.make_async_copy(node_hbm.at[e0t_v.at[ci]], rows_v, sem).wait()

  def compute_scale(ci, rows_v):
    keeps = []
    for g in range(CB // 16):
      a = e0t_v[ci, pl.ds(16 * g, 16)]
      b = e1t_v[ci, pl.ds(16 * g, 16)]
      m = plsc.bitcast(mt_v[ci, pl.ds(16 * g, 16)], _f32)
      enc0 = plsc.load_gather(enc_v, _rowcol(a))
      enc1 = plsc.load_gather(enc_v, _rowcol(b))
      keep = lax.bitwise_and(lax.bitwise_and(enc0, enc1), ones)
      keeps.append((enc0, enc1, keep))
      w = keep.astype(_f32) / (1.0 + jnp.exp(-m))

      @plsc.parallel_loop(0, 16, 1, unroll=2)
      def _scale(r):
        ws = jnp.take_along_axis(w, jnp.full((16,), r, _i32), axis=0)
        for k in range(8):
          rows_v[16 * g + r, pl.ds(16 * k, 16)] = (
              rows_v[16 * g + r, pl.ds(16 * k, 16)] * ws)

    return keeps

  def emit_outputs(ci, keeps):
    kbit = jnp.full((16,), 1 << 28, _i32)
    zi = jnp.zeros((16,), _i32)
    for g in range(CB // 16):
      enc0, enc1, keep = keeps[g]
      kb = keep > 0
      hi = lax.bitwise_or(
          lax.bitwise_or(lax.shift_right_logical(enc0, 1),
                         lax.shift_left(lax.shift_right_logical(enc1, 1), 14)),
          kbit)
      mt_v[ci, pl.ds(16 * g, 16)] = jnp.where(kb, hi, zi)

  def issue_scatter(ci, rows_v, sem):
    pl.delay(100)
    return pltpu.async_copy(rows_v, sh_agg.at[e1t_v.at[ci]], sem, add=True)

  def wait_scatter(ci, rows_v, sem):
    pltpu.make_async_copy(rows_v, sh_agg.at[e1t_v.at[ci]], sem).wait()

  def batch_body(b, carry):
    pltpu.sync_copy(e0r_hbm.at[wid, b], e0t_v)
    pltpu.sync_copy(e1r_hbm.at[wid, b], e1t_v)
    pltpu.sync_copy(mr_hbm.at[wid, b], mt_v)
    issue_gather(0, rows_a, semg_a)

    def pair_body(i, carry2):
      c0 = 2 * i
      c1 = 2 * i + 1
      wait_gather(c0, rows_a, semg_a)

      @pl.when(i > 0)
      def _drain_b():
        wait_scatter(c0 - 1, rows_b, sems_b)

      issue_gather(c1, rows_b, semg_b)
      keeps = compute_scale(c0, rows_a)
      emit_outputs(c0, keeps)
      issue_scatter(c0, rows_a, sems_a)
      wait_gather(c1, rows_b, semg_b)
      wait_scatter(c0, rows_a, sems_a)
      issue_gather(c0 + 2, rows_a, semg_a)
      keeps = compute_scale(c1, rows_b)
      emit_outputs(c1, keeps)
      issue_scatter(c1, rows_b, sems_b)
      return carry2

    lax.fori_loop(0, BCH // 2, pair_body, 0)
    ct = BCH - 1
    wait_gather(ct, rows_a, semg_a)
    wait_scatter(ct - 1, rows_b, sems_b)
    keeps = compute_scale(ct, rows_a)
    emit_outputs(ct, keeps)
    pl.delay(200)
    pltpu.sync_copy(rows_a, sh_agg.at[e1t_v.at[ct]], add=True)
    pltpu.sync_copy(mt_v, pk_out.at[wid, b])
    return carry

  lax.fori_loop(0, NB, batch_body, 0)
  plsc.subcore_barrier()

  @pl.when(t < 10)
  def _export():
    pltpu.sync_copy(sh_agg.at[pl.ds(1000 * t, 1000)],
                    part_out.at[pl.ds(cid * N + 1000 * t, 1000)])


def _comb_body(p_ref, emb_ref, rel_ref, h_ref, t_ref, r_ref, agg_ref, sc_ref):
  agg_ref[...] = p_ref[0] + p_ref[1]
  h = h_ref[0]
  tt = t_ref[0]
  rr = r_ref[0]
  hrow = emb_ref[pl.ds(h, 1), :]
  trow = emb_ref[pl.ds(tt, 1), :]
  rrow = rel_ref[pl.ds(rr, 1), :]
  sc_ref[0, 0] = jnp.sum(hrow * rrow * trow)


_bfs_call = pl.kernel(
    _bfs_body,
    out_type=(
        jax.ShapeDtypeStruct((NR, 128), _i32),
    ),
    mesh=plsc.VectorSubcoreMesh(core_axis_name="c", subcore_axis_name="s"),
    compiler_params=pltpu.CompilerParams(needs_layout_passes=False),
    scratch_types=(
        pltpu.VMEM((NR, 128), _i32),
        pltpu.VMEM((NR, 128), _i32),
        pltpu.VMEM((NR, 128), _i32),
        pltpu.VMEM((NR, 128), _i32),
        pltpu.VMEM((EA,), _i32),
        pltpu.VMEM((EA,), _i32),
        pltpu.VMEM((16,), _i32),
        pltpu.VMEM((8, 128), _i32),
        pltpu.VMEM((NR,), _i32),
        pltpu.VMEM_SHARED((NR, 128), _i32),
    ),
)

_msg_call = pl.kernel(
    _msg_body,
    out_type=(
        jax.ShapeDtypeStruct((NC * NS, NB, BCH, CB), _i32),
        jax.ShapeDtypeStruct((NC * N, D), _f32),
    ),
    mesh=plsc.VectorSubcoreMesh(core_axis_name="c", subcore_axis_name="s"),
    compiler_params=pltpu.CompilerParams(needs_layout_passes=False),
    scratch_types=(
        pltpu.VMEM((NR, 128), _i32),
        pltpu.VMEM((BCH, CB), _i32),
        pltpu.VMEM((BCH, CB), _i32),
        pltpu.VMEM((BCH, CB), _i32),
        pltpu.VMEM((CB, D), _f32),
        pltpu.VMEM((CB, D), _f32),
        pltpu.VMEM_SHARED((N, D), _f32),
        pltpu.SemaphoreType.DMA,
        pltpu.SemaphoreType.DMA,
        pltpu.SemaphoreType.DMA,
        pltpu.SemaphoreType.DMA,
    ),
)

_comb_call = pl.pallas_call(
    _comb_body,
    out_shape=(
        jax.ShapeDtypeStruct((N, D), _f32),
        jax.ShapeDtypeStruct((1, 1), _f32),
    ),
    in_specs=[
        pl.BlockSpec(memory_space=pltpu.VMEM),
        pl.BlockSpec(memory_space=pltpu.VMEM),
        pl.BlockSpec(memory_space=pltpu.VMEM),
        pl.BlockSpec(memory_space=pltpu.SMEM),
        pl.BlockSpec(memory_space=pltpu.SMEM),
        pl.BlockSpec(memory_space=pltpu.SMEM),
    ],
    out_specs=(
        pl.BlockSpec(memory_space=pltpu.VMEM),
        pl.BlockSpec(memory_space=pltpu.SMEM),
    ),
)


def kernel(node_emb, rel_emb, edge_mask, edge_index, edge_type,
           head_idx, tail_idx, rel_idx):
  del edge_type
  e0 = edge_index[0]
  e1 = edge_index[1]
  h = jnp.asarray(head_idx, _i32)
  t = jnp.asarray(tail_idx, _i32)
  r = jnp.asarray(rel_idx, _i32)
  init_idx = jnp.concatenate(
      [h[None], t[None], jnp.broadcast_to(h[None], (14,))])
  zero_i = jnp.zeros((NR, 128), _i32)
  zero_f = jnp.zeros((1000, D), _f32)

  (enc2d,) = _bfs_call(e0, e1, init_idx, zero_i)
  e0r = e0.reshape(NC * NS, NB, BCH, CB)
  e1r = e1.reshape(NC * NS, NB, BCH, CB)
  mr = lax.bitcast_convert_type(edge_mask, _i32).reshape(NC * NS, NB, BCH, CB)
  pk, parts = _msg_call(node_emb, e0r, e1r, mr, enc2d, zero_f)
  p = parts.reshape(NC, N, D)
  agg, score = _comb_call(p, node_emb, rel_emb, h[None], t[None], r[None])
  pk = pk.reshape(E)
  sub_src = (pk & 0x3FFF) - 1
  sub_dst = ((pk >> 14) & 0x3FFF) - 1
  keep = (pk >> 28).astype(bool)
  return (agg, score.reshape(()), sub_src, sub_dst, keep)
